# baseline (speedup 1.0000x reference)
import functools

import jax
import jax.numpy as jnp
from jax import lax
from jax.experimental import pallas as pl
from jax.experimental.pallas import tpu as pltpu
from jax.experimental.pallas import tpu_sc as plsc

_N = 1048576
_NW = 32
_CHUNK = _N // _NW
_SUB = 8192
_NSUB = _CHUNK // _SUB

_mesh = plsc.VectorSubcoreMesh(core_axis_name="c", subcore_axis_name="s")


@functools.partial(
    pl.kernel,
    mesh=_mesh,
    out_type=[
        jax.ShapeDtypeStruct((_N,), jnp.float32),
        jax.ShapeDtypeStruct((3, _N), jnp.float32),
    ],
    scratch_types=[
        pltpu.VMEM((3, _SUB), jnp.float32),
        pltpu.VMEM((1, _SUB), jnp.float32),
        pltpu.VMEM((1, _SUB), jnp.float32),
    ],
)
def _sc_balls(pos_hbm, den_hbm, rad_hbm, pos_v, mask_v, zero_v):
    wid = lax.axis_index("s") * 2 + lax.axis_index("c")
    base = wid * _CHUNK

    def zinit(i, carry):
        zero_v[0, pl.ds(i * 16, 16)] = jnp.zeros((16,), jnp.float32)
        return carry

    lax.fori_loop(0, _SUB // 16, zinit, 0)

    def body(j, carry):
        off = base + j * _SUB
        pltpu.sync_copy(pos_hbm.at[:, pl.ds(off, _SUB)], pos_v)

        def inner(i, c):
            s = pl.ds(i * 16, 16)
            x = pos_v[0, s]
            y = pos_v[1, s]
            z = pos_v[2, s]
            zz = z * z
            dx1 = x - 0.5
            dx2 = x + 0.5
            dy2 = y + 0.2
            q1 = (dx1 * dx1 + y * y) + zz
            q2 = (dx2 * dx2 + dy2 * dy2) + zz
            inside = (q1 < 0.3) | (q2 < 0.8)
            mask_v[0, s] = jnp.where(inside, jnp.float32(1.0), jnp.float32(0.0))
            return c

        lax.fori_loop(0, _SUB // 16, inner, 0)
        pltpu.sync_copy(mask_v.at[0], den_hbm.at[pl.ds(off, _SUB)])
        pltpu.sync_copy(mask_v, rad_hbm.at[pl.ds(0, 1), pl.ds(off, _SUB)])
        pltpu.sync_copy(zero_v, rad_hbm.at[pl.ds(1, 1), pl.ds(off, _SUB)])
        pltpu.sync_copy(zero_v, rad_hbm.at[pl.ds(2, 1), pl.ds(off, _SUB)])
        return carry

    lax.fori_loop(0, _NSUB, body, 0)


@jax.jit
def _run(position):
    n = position.shape[0]
    pos_t = position.T
    den, rad = _sc_balls(pos_t)
    return den, rad.T


def kernel(position, direction):
    del direction
    return _run(position)


# speedup vs baseline: 3.7666x; 3.7666x over previous
import functools

import jax
import jax.numpy as jnp
from jax.experimental import pallas as pl
from jax.experimental.pallas import tpu as pltpu

_L = 131072
_GRID = 8


def _balls_kernel(pos_hbm, den_ref, rad_ref, buf, sem):
    i = pl.program_id(0)
    slot = jax.lax.rem(i, 2)
    nxt = jax.lax.rem(i + 1, 2)

    @pl.when(i == 0)
    def _():
        pltpu.make_async_copy(
            pos_hbm.at[:, pl.ds(0, _L)], buf.at[0], sem.at[0]
        ).start()

    @pl.when(i + 1 < _GRID)
    def _():
        pltpu.make_async_copy(
            pos_hbm.at[:, pl.ds((i + 1) * _L, _L)], buf.at[nxt], sem.at[nxt]
        ).start()

    pltpu.make_async_copy(
        pos_hbm.at[:, pl.ds(i * _L, _L)], buf.at[slot], sem.at[slot]
    ).wait()

    x = buf[slot, 0:1, :]
    y = buf[slot, 1:2, :]
    z = buf[slot, 2:3, :]

    zz = z * z
    q1 = (jnp.square(x - 0.5) + jnp.square(y)) + zz
    q2 = (jnp.square(x + 0.5) + jnp.square(y + 0.2)) + zz
    inside = (q1 < 0.3) | (q2 < 0.8)

    m = jnp.where(inside, jnp.float32(1.0), jnp.float32(0.0))
    den_ref[...] = m
    rad_ref[0:1, :] = m
    rad_ref[1:3, :] = jnp.zeros((2, _L), jnp.float32)


@jax.jit
def _run(position):
    n = position.shape[0]
    pos_t = position.T
    den, rad = pl.pallas_call(
        _balls_kernel,
        grid=(_GRID,),
        in_specs=[pl.BlockSpec(memory_space=pl.ANY)],
        out_specs=[
            pl.BlockSpec((1, _L), lambda i: (0, i)),
            pl.BlockSpec((3, _L), lambda i: (0, i)),
        ],
        out_shape=[
            jax.ShapeDtypeStruct((1, n), jnp.float32),
            jax.ShapeDtypeStruct((3, n), jnp.float32),
        ],
        scratch_shapes=[
            pltpu.VMEM((2, 3, _L), jnp.float32),
            pltpu.SemaphoreType.DMA((2,)),
        ],
    )(pos_t)
    return den.reshape(n), rad.T


def kernel(position, direction):
    del direction
    return _run(position)


# speedup vs baseline: 3.9496x; 1.0486x over previous
import functools

import jax
import jax.numpy as jnp
from jax.experimental import pallas as pl
from jax.experimental.pallas import tpu as pltpu

_L = 262144
_GRID = 4


def _balls_kernel(pos_hbm, den_ref, rad_ref, buf, sem):
    i = pl.program_id(0)
    slot = jax.lax.rem(i, 2)
    nxt = jax.lax.rem(i + 1, 2)

    @pl.when(i == 0)
    def _():
        pltpu.make_async_copy(
            pos_hbm.at[:, pl.ds(0, _L)], buf.at[0], sem.at[0]
        ).start()

    @pl.when(i + 1 < _GRID)
    def _():
        pltpu.make_async_copy(
            pos_hbm.at[:, pl.ds((i + 1) * _L, _L)], buf.at[nxt], sem.at[nxt]
        ).start()

    pltpu.make_async_copy(
        pos_hbm.at[:, pl.ds(i * _L, _L)], buf.at[slot], sem.at[slot]
    ).wait()

    x = buf[slot, 0:1, :]
    y = buf[slot, 1:2, :]
    z = buf[slot, 2:3, :]

    zz = z * z
    q1 = (jnp.square(x - 0.5) + jnp.square(y)) + zz
    q2 = (jnp.square(x + 0.5) + jnp.square(y + 0.2)) + zz
    inside = (q1 < 0.3) | (q2 < 0.8)

    m = jnp.where(inside, jnp.float32(1.0), jnp.float32(0.0))
    den_ref[...] = m
    rad_ref[0:1, :] = m
    rad_ref[1:3, :] = jnp.zeros((2, _L), jnp.float32)


@jax.jit
def _run(position):
    n = position.shape[0]
    pos_t = position.T
    den, rad = pl.pallas_call(
        _balls_kernel,
        grid=(_GRID,),
        in_specs=[pl.BlockSpec(memory_space=pl.ANY)],
        out_specs=[
            pl.BlockSpec((1, _L), lambda i: (0, i)),
            pl.BlockSpec((3, _L), lambda i: (0, i)),
        ],
        out_shape=[
            jax.ShapeDtypeStruct((1, n), jnp.float32),
            jax.ShapeDtypeStruct((3, n), jnp.float32),
        ],
        scratch_shapes=[
            pltpu.VMEM((2, 3, _L), jnp.float32),
            pltpu.SemaphoreType.DMA((2,)),
        ],
    )(pos_t)
    return den.reshape(n), rad.T


def kernel(position, direction):
    del direction
    return _run(position)


# speedup vs baseline: 4.4446x; 1.1253x over previous
import functools

import jax
import jax.numpy as jnp
from jax.experimental import pallas as pl
from jax.experimental.pallas import tpu as pltpu

_L = 524288
_GRID = 2


def _balls_kernel(pos_hbm, den_ref, rad_ref, buf, sem):
    i = pl.program_id(0)
    slot = jax.lax.rem(i, 2)
    nxt = jax.lax.rem(i + 1, 2)

    @pl.when(i == 0)
    def _():
        pltpu.make_async_copy(
            pos_hbm.at[:, pl.ds(0, _L)], buf.at[0], sem.at[0]
        ).start()

    @pl.when(i + 1 < _GRID)
    def _():
        pltpu.make_async_copy(
            pos_hbm.at[:, pl.ds((i + 1) * _L, _L)], buf.at[nxt], sem.at[nxt]
        ).start()

    pltpu.make_async_copy(
        pos_hbm.at[:, pl.ds(i * _L, _L)], buf.at[slot], sem.at[slot]
    ).wait()

    x = buf[slot, 0:1, :]
    y = buf[slot, 1:2, :]
    z = buf[slot, 2:3, :]

    zz = z * z
    q1 = (jnp.square(x - 0.5) + jnp.square(y)) + zz
    q2 = (jnp.square(x + 0.5) + jnp.square(y + 0.2)) + zz
    inside = (q1 < 0.3) | (q2 < 0.8)

    m = jnp.where(inside, jnp.float32(1.0), jnp.float32(0.0))
    den_ref[...] = m
    rad_ref[0:1, :] = m
    rad_ref[1:3, :] = jnp.zeros((2, _L), jnp.float32)


@jax.jit
def _run(position):
    n = position.shape[0]
    pos_t = position.T
    den, rad = pl.pallas_call(
        _balls_kernel,
        grid=(_GRID,),
        in_specs=[pl.BlockSpec(memory_space=pl.ANY)],
        out_specs=[
            pl.BlockSpec((1, _L), lambda i: (0, i)),
            pl.BlockSpec((3, _L), lambda i: (0, i)),
        ],
        out_shape=[
            jax.ShapeDtypeStruct((1, n), jnp.float32),
            jax.ShapeDtypeStruct((3, n), jnp.float32),
        ],
        scratch_shapes=[
            pltpu.VMEM((2, 3, _L), jnp.float32),
            pltpu.SemaphoreType.DMA((2,)),
        ],
    )(pos_t)
    return den.reshape(n), rad.T


def kernel(position, direction):
    del direction
    return _run(position)
